# baseline (speedup 1.0000x reference)
import functools

import jax
import jax.numpy as jnp
from jax import lax
from jax.experimental import pallas as pl
from jax.experimental.pallas import tpu as pltpu
from jax.experimental.pallas import tpu_sc as plsc

_EMBED_DIM = 64
_CHUNK = 800


def _make_gather(n_rows: int):
  info = plsc.get_sparse_core_info()
  nw = info.num_cores * info.num_subcores
  b_per_w = n_rows // nw
  n_chunks = b_per_w // _CHUNK
  n_pairs = n_chunks // 2
  assert n_chunks * _CHUNK == b_per_w and n_pairs * 2 == n_chunks
  mesh = plsc.VectorSubcoreMesh(core_axis_name="c", subcore_axis_name="s")

  @functools.partial(
      pl.kernel,
      mesh=mesh,
      out_type=jax.ShapeDtypeStruct((n_rows, _EMBED_DIM), jnp.float32),
      scratch_types=[
          pltpu.VMEM((_CHUNK,), jnp.int32),
          pltpu.VMEM((_CHUNK,), jnp.int32),
          pltpu.VMEM((_CHUNK, _EMBED_DIM), jnp.float32),
          pltpu.VMEM((_CHUNK, _EMBED_DIM), jnp.float32),
          pltpu.SemaphoreType.DMA,
          pltpu.SemaphoreType.DMA,
          pltpu.SemaphoreType.DMA,
          pltpu.SemaphoreType.DMA,
      ],
      compiler_params=pltpu.CompilerParams(use_tc_tiling_on_sc=False),
  )
  def gather(idx_hbm, table_hbm, out_hbm, iv0, iv1, rv0, rv1, si0, si1, sg0,
             sg1):
    wid = lax.axis_index("s") * info.num_cores + lax.axis_index("c")
    base = wid * b_per_w
    idx_v = (iv0, iv1)
    rows_v = (rv0, rv1)
    sem_i = (si0, si1)
    sem_g = (sg0, sg1)

    def idx_start(g, b):
      pltpu.async_copy(
          idx_hbm.at[pl.ds(base + g * _CHUNK, _CHUNK)], idx_v[b], sem_i[b])

    def idx_wait(b):
      pltpu.make_async_copy(
          idx_hbm.at[pl.ds(base, _CHUNK)], idx_v[b], sem_i[b]).wait()

    def gather_start(b):
      pltpu.async_copy(table_hbm.at[idx_v[b]], rows_v[b], sem_g[b])

    def gather_wait(b):
      pltpu.make_async_copy(
          table_hbm.at[idx_v[b]], rows_v[b], sem_g[b]).wait()

    def store(g, b):
      pltpu.sync_copy(rows_v[b], out_hbm.at[pl.ds(base + g * _CHUNK, _CHUNK)])

    idx_start(0, 0)
    idx_start(1, 1)
    idx_wait(0)
    gather_start(0)

    def pair_body(p, carry):
      g = 2 * p
      for b in (0, 1):
        gather_wait(b)
        idx_wait(1 - b)
        gather_start(1 - b)
        store(g + b, b)
        idx_start(g + b + 2, b)
      return carry

    lax.fori_loop(0, n_pairs - 1, pair_body, 0)

    g = n_chunks - 2
    gather_wait(0)
    idx_wait(1)
    gather_start(1)
    store(g, 0)
    gather_wait(1)
    store(g + 1, 1)

  return gather


def kernel(input_ids, aux, table):
  idx = input_ids.reshape(-1).astype(jnp.int32)
  half = idx.shape[0] // 2
  g = _make_gather(half)
  o1 = g(idx[:half], table)
  o2 = g(idx[half:], table)
  bh = input_ids.shape[0] // 2
  out = jnp.concatenate(
      [o1.reshape((bh,) + input_ids.shape[1:] + (_EMBED_DIM,)),
       o2.reshape((bh,) + input_ids.shape[1:] + (_EMBED_DIM,))], axis=0)
  return (out,)


# speedup vs baseline: 1.0569x; 1.0569x over previous
import functools

import jax
import jax.numpy as jnp
from jax import lax
from jax.experimental import pallas as pl
from jax.experimental.pallas import tpu as pltpu
from jax.experimental.pallas import tpu_sc as plsc

_EMBED_DIM = 64
_CHUNK = 800


def _make_gather(n_rows: int):
  info = plsc.get_sparse_core_info()
  nw = info.num_cores * info.num_subcores
  b_per_w = n_rows // nw
  n_chunks = b_per_w // _CHUNK
  n_pairs = n_chunks // 2
  assert n_chunks * _CHUNK == b_per_w and n_pairs * 2 == n_chunks
  mesh = plsc.VectorSubcoreMesh(core_axis_name="c", subcore_axis_name="s")

  @functools.partial(
      pl.kernel,
      mesh=mesh,
      out_type=jax.ShapeDtypeStruct((n_rows, _EMBED_DIM), jnp.float32),
      scratch_types=[
          pltpu.VMEM((_CHUNK,), jnp.int32),
          pltpu.VMEM((_CHUNK,), jnp.int32),
          pltpu.VMEM((_CHUNK, _EMBED_DIM), jnp.float32),
          pltpu.VMEM((_CHUNK, _EMBED_DIM), jnp.float32),
          pltpu.SemaphoreType.DMA,
          pltpu.SemaphoreType.DMA,
          pltpu.SemaphoreType.DMA,
          pltpu.SemaphoreType.DMA,
      ],
      compiler_params=pltpu.CompilerParams(use_tc_tiling_on_sc=False),
  )
  def gather(idx_hbm, table_hbm, out_hbm, iv0, iv1, rv0, rv1, si0, si1, sg0,
             sg1):
    wid = lax.axis_index("s") * info.num_cores + lax.axis_index("c")
    base = wid * b_per_w
    idx_v = (iv0, iv1)
    rows_v = (rv0, rv1)
    sem_i = (si0, si1)
    sem_g = (sg0, sg1)

    def idx_start(g, b):
      pltpu.async_copy(
          idx_hbm.at[pl.ds(base + g * _CHUNK, _CHUNK)], idx_v[b], sem_i[b])

    def idx_wait(b):
      pltpu.make_async_copy(
          idx_hbm.at[pl.ds(base, _CHUNK)], idx_v[b], sem_i[b]).wait()

    def gather_start(b):
      pltpu.async_copy(table_hbm.at[idx_v[b]], rows_v[b], sem_g[b])

    def gather_wait(b):
      pltpu.make_async_copy(
          table_hbm.at[idx_v[b]], rows_v[b], sem_g[b]).wait()

    def store(g, b):
      pltpu.sync_copy(rows_v[b], out_hbm.at[pl.ds(base + g * _CHUNK, _CHUNK)])

    idx_start(0, 0)
    idx_start(1, 1)
    idx_wait(0)
    gather_start(0)

    def pair_body(p, carry):
      g = 2 * p
      for b in (0, 1):
        gather_wait(b)
        idx_wait(1 - b)
        gather_start(1 - b)
        store(g + b, b)
        idx_start(g + b + 2, b)
      return carry

    lax.fori_loop(0, n_pairs - 1, pair_body, 0)

    g = n_chunks - 2
    gather_wait(0)
    idx_wait(1)
    gather_start(1)
    store(g, 0)
    gather_wait(1)
    store(g + 1, 1)

  return gather


def kernel(input_ids, aux, table):
  idx = input_ids.reshape(-1).astype(jnp.int32)
  out = _make_gather(idx.shape[0])(idx, table)
  return (out.reshape(input_ids.shape + (_EMBED_DIM,)),)
